# baseline (speedup 1.0000x reference)
import jax
import jax.numpy as jnp
from jax import lax
from jax.experimental import pallas as pl
from jax.experimental.pallas import tpu as pltpu
from jax.experimental.pallas import tpu_sc as plsc

N_SEG = 10000
NNZ = 320000
D = 128
NCLS = 40

NC, NS = 2, 16
NW = NC * NS
CPT = D // NW
K = 4000
NCH = NNZ // K
GPC = K // 16


def _sc_prop_body(h_hbm, src_hbm, dst_hbm, z_hbm, out_hbm,
                  tb, ac, sbuf, dbuf, se0, se1, de0, de1):
    c = lax.axis_index("c")
    s = lax.axis_index("s")
    t = s * NC + c

    for j in range(CPT):
        pltpu.sync_copy(h_hbm.at[t, j], tb.at[j])
        pltpu.sync_copy(z_hbm, ac.at[j])

    def _stage(i, b, ses, des):
        off = pl.multiple_of(i * K, 8)
        pltpu.async_copy(src_hbm.at[pl.ds(off, K)], sbuf.at[b], ses)
        pltpu.async_copy(dst_hbm.at[pl.ds(off, K)], dbuf.at[b], des)

    def _wait_stage(b, ses, des):
        pltpu.make_async_copy(src_hbm.at[pl.ds(0, K)], sbuf.at[b], ses).wait()
        pltpu.make_async_copy(dst_hbm.at[pl.ds(0, K)], dbuf.at[b], des).wait()

    def _consume(b):
        def grp(g, carry):
            off = pl.multiple_of(g * 16, 8)
            si = sbuf[b, pl.ds(off, 16)]
            di = dbuf[b, pl.ds(off, 16)]
            for j in range(CPT):
                v = plsc.load_gather(tb.at[j], [si])
                plsc.addupdate_scatter(ac.at[j], [di], v)
            return carry
        lax.fori_loop(0, GPC, grp, 0)

    _stage(0, 0, se0, de0)

    def pair(p, carry):
        i0 = 2 * p
        _stage(i0 + 1, 1, se1, de1)
        _wait_stage(0, se0, de0)
        _consume(0)

        @pl.when(i0 + 2 < NCH)
        def _():
            _stage(i0 + 2, 0, se0, de0)
        _wait_stage(1, se1, de1)
        _consume(1)
        return carry

    lax.fori_loop(0, NCH // 2, pair, 0)

    for j in range(CPT):
        pltpu.sync_copy(ac.at[j], out_hbm.at[t, j])


_sc_propagate = pl.kernel(
    _sc_prop_body,
    out_type=jax.ShapeDtypeStruct((NW, CPT, N_SEG), jnp.float32),
    mesh=plsc.VectorSubcoreMesh(core_axis_name="c", subcore_axis_name="s"),
    compiler_params=pltpu.CompilerParams(use_tc_tiling_on_sc=False, needs_layout_passes=False),
    scratch_types=[
        pltpu.VMEM((CPT, N_SEG), jnp.float32),
        pltpu.VMEM((CPT, N_SEG), jnp.float32),
        pltpu.VMEM((2, K), jnp.int32),
        pltpu.VMEM((2, K), jnp.int32),
        pltpu.SemaphoreType.DMA,
        pltpu.SemaphoreType.DMA,
        pltpu.SemaphoreType.DMA,
        pltpu.SemaphoreType.DMA,
    ],
)


def _sc_cnt_body(dst_hbm, cnt_hbm, hist, dbuf):
    c = lax.axis_index("c")
    s = lax.axis_index("s")
    t = s * NC + c

    pltpu.sync_copy(dst_hbm.at[t], dbuf)
    zeros = jnp.zeros((16,), jnp.float32)

    def zgrp(g, carry):
        hist[pl.ds(pl.multiple_of(g * 16, 8), 16)] = zeros
        return carry

    lax.fori_loop(0, N_SEG // 16, zgrp, 0)

    ones = jnp.full((16,), 1.0, jnp.float32)

    def grp(g, carry):
        off = pl.multiple_of(g * 16, 8)
        di = dbuf[pl.ds(off, 16)]
        plsc.addupdate_scatter(hist, [di], ones)
        return carry

    lax.fori_loop(0, N_SEG // 16, grp, 0)
    pltpu.sync_copy(hist, cnt_hbm.at[t])


_sc_counts = pl.kernel(
    _sc_cnt_body,
    out_type=jax.ShapeDtypeStruct((NW, N_SEG), jnp.float32),
    mesh=plsc.VectorSubcoreMesh(core_axis_name="c", subcore_axis_name="s"),
    compiler_params=pltpu.CompilerParams(use_tc_tiling_on_sc=False, needs_layout_passes=False),
    scratch_types=[
        pltpu.VMEM((N_SEG,), jnp.float32),
        pltpu.VMEM((N_SEG,), jnp.int32),
    ],
)


R = 2048
GRID = (N_SEG + R - 1) // R

_HI = jax.lax.Precision.HIGHEST


def _dot(a, b):
    return jax.lax.dot_general(a, b, (((1,), (0,)), ((), ())),
                               precision=_HI,
                               preferred_element_type=jnp.float32)


def _store_t(o_ref, g):
    o_ref[...] = g.T.reshape(NW, CPT, g.shape[0])


def _enc_body(x_ref, w1, b1, w2, b2, o_ref):
    t = jnp.maximum(_dot(x_ref[...], w1[...]) + b1[...], 0.0)
    _store_t(o_ref, jnp.maximum(_dot(t, w2[...]) + b2[...], 0.0))


def _agg(s_ref, c_ref):
    cnt = jnp.sum(c_ref[...], axis=0, keepdims=True)
    inv = (1.0 / jnp.maximum(cnt, 1.0)).T
    sums = s_ref[...].reshape(D, -1).T
    return sums * inv


def _mid_body(s_ref, c_ref, wd1, bd1, wd2, bd2, we1, be1, we2, be2, o_ref):
    t = _agg(s_ref, c_ref)
    t = jnp.maximum(_dot(t, wd1[...]) + bd1[...], 0.0)
    t = jnp.maximum(_dot(t, wd2[...]) + bd2[...], 0.0)
    t = jnp.maximum(_dot(t, we1[...]) + be1[...], 0.0)
    _store_t(o_ref, jnp.maximum(_dot(t, we2[...]) + be2[...], 0.0))


def _fin_body(s_ref, c_ref, wd1, bd1, wd2, bd2, wc1, bc1, wc2, bc2, o_ref):
    t = _agg(s_ref, c_ref)
    t = jnp.maximum(_dot(t, wd1[...]) + bd1[...], 0.0)
    t = jnp.maximum(_dot(t, wd2[...]) + bd2[...], 0.0)
    t = jnp.maximum(_dot(t, wc1[...]) + bc1[...], 0.0)
    o_ref[...] = _dot(t, wc2[...]) + bc2[...]


def _wspec(shape):
    return pl.BlockSpec(shape, lambda i: (0,) * len(shape))


_T_OUT = pl.BlockSpec((NW, CPT, R), lambda i: (0, 0, i))
_T_SHAPE = jax.ShapeDtypeStruct((NW, CPT, N_SEG), jnp.float32)


def _make_enc():
    return pl.pallas_call(
        _enc_body,
        grid=(GRID,),
        in_specs=[pl.BlockSpec((R, D), lambda i: (i, 0)),
                  _wspec((D, D)), _wspec((1, D)), _wspec((D, D)), _wspec((1, D))],
        out_specs=_T_OUT,
        out_shape=_T_SHAPE,
    )


def _make_mid(body, final):
    wspecs = []
    for _ in range(3):
        wspecs += [_wspec((D, D)), _wspec((1, D))]
    out_cols = NCLS if final else D
    wspecs += [_wspec((D, out_cols)), _wspec((1, out_cols))]
    return pl.pallas_call(
        body,
        grid=(GRID,),
        in_specs=[pl.BlockSpec((NW, CPT, R), lambda i: (0, 0, i)),
                  pl.BlockSpec((NW, R), lambda i: (0, i))] + wspecs,
        out_specs=(pl.BlockSpec((R, NCLS), lambda i: (i, 0)) if final
                   else _T_OUT),
        out_shape=(jax.ShapeDtypeStruct((N_SEG, NCLS), jnp.float32) if final
                   else _T_SHAPE),
    )


_enc_call = _make_enc()
_mid_call = _make_mid(_mid_body, final=False)
_fin_call = _make_mid(_fin_body, final=True)


def _unpack(layers):
    (w1, b1), (w2, b2) = layers
    return w1, b1.reshape(1, -1), w2, b2.reshape(1, -1)


def kernel(x, edge_index, params):
    src = edge_index[0]
    dst = edge_index[1]
    src_rs = src.reshape(NW, N_SEG)
    dst_rs = dst.reshape(NW, N_SEG)
    z = jnp.zeros((N_SEG,), jnp.float32)

    cnt_h = _sc_counts(dst_rs)
    cnt_n = _sc_counts(src_rs)

    g = _enc_call(x, *_unpack(params["V2E"][0]["enc"]))

    s0 = _sc_propagate(g, src, dst, z)
    g = _mid_call(s0, cnt_h, *_unpack(params["V2E"][0]["dec"]),
                  *_unpack(params["E2V"][0]["enc"]))

    s1 = _sc_propagate(g, dst, src, z)
    g = _mid_call(s1, cnt_n, *_unpack(params["E2V"][0]["dec"]),
                  *_unpack(params["V2E"][1]["enc"]))

    s2 = _sc_propagate(g, src, dst, z)
    g = _mid_call(s2, cnt_h, *_unpack(params["V2E"][1]["dec"]),
                  *_unpack(params["E2V"][1]["enc"]))

    s3 = _sc_propagate(g, dst, src, z)
    out = _fin_call(s3, cnt_n, *_unpack(params["E2V"][1]["dec"]),
                    *_unpack(params["clf"]))
    return out
